# baseline (speedup 1.0000x reference)
import functools

import jax
import jax.numpy as jnp
from jax import lax
from jax.experimental import pallas as pl
from jax.experimental.pallas import tpu as pltpu
from jax.experimental.pallas import tpu_sc as plsc

_K = 1024
_NROWS = 128
_NCOLS = 32768
_NCHUNK = _NCOLS // 16
_NBINS = 32768
_NCAND = 32768
_ROWS_PER_WORKER = _NROWS // 32


def _sc_body(x_hbm, out_hbm, x_v, hist_v, cand_v):
    sign = jnp.int32(-2**31)
    inv = jnp.int32(0x7FFFFFFF)
    lanes = lax.iota(jnp.int32, 16)
    ones = jnp.ones((16,), jnp.int32)
    wid = lax.axis_index("s") * 2 + lax.axis_index("c")

    @plsc.parallel_loop(0, _NBINS // 16, unroll=8)
    def zero_loop(i):
        hist_v[pl.ds(i * 16, 16)] = jnp.zeros((16,), jnp.int32)

    def do_row(r, _):
        row = wid * _ROWS_PER_WORKER + r
        pltpu.sync_copy(x_hbm.at[row], x_v)

        @plsc.parallel_loop(0, _NCHUNK, unroll=16,
                            carry=jnp.full((16,), -2**31, jnp.int32))
        def hist_loop(i, mx):
            bits = plsc.bitcast(x_v[pl.ds(i * 16, 16)], jnp.int32)
            xk = jnp.where(bits < 0, bits ^ inv, bits)
            bins = lax.shift_right_logical(xk ^ sign, 17)
            plsc.addupdate_scatter(hist_v, (bins,), ones)
            return jnp.maximum(mx, xk)

        mx = jnp.max(hist_loop)
        vj0 = lax.shift_right_logical(mx ^ sign, 17) // 16

        def scan_cond(c):
            return (c[3] == 0) & (c[1] >= 0)

        def scan_body(c):
            rs, vj, b, _ = c
            v = hist_v[pl.ds(vj * 16, 16)]
            rev = lax.rev(v, (0,))
            cs = jnp.cumsum(rev)
            crossed = (rs + cs) >= _K
            anyc = jnp.max(crossed.astype(jnp.int32))
            istar = jnp.max(plsc.all_reduce_ffs(crossed))
            before = jnp.sum(jnp.where(lanes < istar, rev, 0))
            b_new = vj * 16 + 15 - istar
            rs_new = jnp.where(anyc == 1, rs + before, rs + cs[15])
            return (rs_new, vj - 1,
                    jnp.where(anyc == 1, b_new, b), anyc)

        cnt_above, _, b_star, _ = lax.while_loop(
            scan_cond, scan_body,
            (jnp.int32(0), vj0, jnp.int32(0), jnp.int32(0)))
        rprime = _K - cnt_above

        @plsc.parallel_loop(0, _NCHUNK, unroll=8, carry=jnp.int32(0))
        def col_loop(i, off):
            bits = plsc.bitcast(x_v[pl.ds(i * 16, 16)], jnp.int32)
            xk = jnp.where(bits < 0, bits ^ inv, bits)
            bins = lax.shift_right_logical(xk ^ sign, 17)
            m = bins == b_star
            mi = m.astype(jnp.int32)
            pos = off + jnp.cumsum(mi) - 1
            plsc.store_scatter(cand_v, (pos,), xk, mask=m)
            return off + jnp.sum(mi)

        m_count = col_loop
        nv = (m_count + 15) >> 4

        def bit_body(bi, p):
            t = p | lax.shift_left(jnp.int32(1), 16 - bi)
            txi = (lax.shift_left(b_star, 17) | t) ^ sign

            def cnt_body(vj, acc):
                cv = cand_v[pl.ds(vj * 16, 16)]
                inb = (vj * 16 + lanes) < m_count
                return acc + jnp.where(inb & (cv >= txi), 1, 0)

            accv = lax.fori_loop(0, nv, cnt_body, jnp.zeros((16,), jnp.int32))
            return jnp.where(jnp.sum(accv) >= rprime, t, p)

        p_low = lax.fori_loop(0, 17, bit_body, jnp.int32(0))
        tx = (lax.shift_left(b_star, 17) | p_low) ^ sign

        @plsc.parallel_loop(0, _NCHUNK, unroll=16)
        def mask_loop(i):
            bits = plsc.bitcast(x_v[pl.ds(i * 16, 16)], jnp.int32)
            xk = jnp.where(bits < 0, bits ^ inv, bits)
            x_v[pl.ds(i * 16, 16)] = jnp.where(xk >= tx,
                                               jnp.float32(1.0),
                                               jnp.float32(0.0))
            hist_v[pl.ds(i * 16, 16)] = jnp.zeros((16,), jnp.int32)

        pltpu.sync_copy(x_v, out_hbm.at[row])
        return _

    lax.fori_loop(0, _ROWS_PER_WORKER, do_row, jnp.int32(0))


@jax.jit
def kernel(output_a):
    mesh = plsc.VectorSubcoreMesh(core_axis_name="c", subcore_axis_name="s",
                                  num_cores=2, num_subcores=16)
    fn = functools.partial(
        pl.kernel,
        out_type=jax.ShapeDtypeStruct((_NROWS, _NCOLS), jnp.float32),
        mesh=mesh,
        compiler_params=pltpu.CompilerParams(needs_layout_passes=False),
        scratch_types=[
            pltpu.VMEM((_NCOLS,), jnp.float32),
            pltpu.VMEM((_NBINS,), jnp.int32),
            pltpu.VMEM((_NCAND,), jnp.int32),
        ],
    )(_sc_body)
    return fn(output_a)


# speedup vs baseline: 1.0022x; 1.0022x over previous
import functools

import jax
import jax.numpy as jnp
from jax import lax
from jax.experimental import pallas as pl
from jax.experimental.pallas import tpu as pltpu
from jax.experimental.pallas import tpu_sc as plsc

_K = 1024
_NROWS = 128
_NCOLS = 32768
_NCHUNK = _NCOLS // 16
_NBINS = 32768
_NCAND = 32768
_ROWS_PER_WORKER = _NROWS // 32


def _sc_body(x_hbm, out_hbm, x_v, hist_v, cand_v):
    sign = jnp.int32(-2**31)
    inv = jnp.int32(0x7FFFFFFF)
    lanes = lax.iota(jnp.int32, 16)
    ones = jnp.ones((16,), jnp.int32)
    wid = lax.axis_index("s") * 2 + lax.axis_index("c")

    @plsc.parallel_loop(0, _NBINS // 16, unroll=8)
    def zero_loop(i):
        hist_v[pl.ds(i * 16, 16)] = jnp.zeros((16,), jnp.int32)

    def do_row(r, _):
        row = wid * _ROWS_PER_WORKER + r
        pltpu.sync_copy(x_hbm.at[row], x_v)

        @plsc.parallel_loop(0, _NCHUNK, unroll=8,
                            carry=jnp.full((16,), -2**31, jnp.int32))
        def hist_loop(i, mx):
            bits = plsc.bitcast(x_v[pl.ds(i * 16, 16)], jnp.int32)
            xk = jnp.where(bits < 0, bits ^ inv, bits)
            bins = lax.shift_right_logical(xk ^ sign, 17)
            plsc.addupdate_scatter(hist_v, (bins,), ones)
            return jnp.maximum(mx, xk)

        mx = jnp.max(hist_loop)
        vj0 = lax.shift_right_logical(mx ^ sign, 17) // 16

        def scan_cond(c):
            return (c[3] == 0) & (c[1] >= 0)

        def scan_body(c):
            rs, vj, b, _ = c
            v = hist_v[pl.ds(vj * 16, 16)]
            rev = lax.rev(v, (0,))
            cs = jnp.cumsum(rev)
            crossed = (rs + cs) >= _K
            anyc = jnp.max(crossed.astype(jnp.int32))
            istar = jnp.max(plsc.all_reduce_ffs(crossed))
            before = jnp.sum(jnp.where(lanes < istar, rev, 0))
            b_new = vj * 16 + 15 - istar
            rs_new = jnp.where(anyc == 1, rs + before, rs + cs[15])
            return (rs_new, vj - 1,
                    jnp.where(anyc == 1, b_new, b), anyc)

        cnt_above, _, b_star, _ = lax.while_loop(
            scan_cond, scan_body,
            (jnp.int32(0), vj0, jnp.int32(0), jnp.int32(0)))
        rprime = _K - cnt_above

        @plsc.parallel_loop(0, _NCHUNK, unroll=8, carry=jnp.int32(0))
        def col_loop(i, off):
            bits = plsc.bitcast(x_v[pl.ds(i * 16, 16)], jnp.int32)
            xk = jnp.where(bits < 0, bits ^ inv, bits)
            bins = lax.shift_right_logical(xk ^ sign, 17)
            m = bins == b_star
            mi = m.astype(jnp.int32)
            pos = off + jnp.cumsum(mi) - 1
            plsc.store_scatter(cand_v, (pos,), xk, mask=m)
            return off + jnp.sum(mi)

        m_count = col_loop
        nv = (m_count + 15) >> 4

        def bit_body(bi, p):
            t = p | lax.shift_left(jnp.int32(1), 16 - bi)
            txi = (lax.shift_left(b_star, 17) | t) ^ sign

            def cnt_body(vj, acc):
                cv = cand_v[pl.ds(vj * 16, 16)]
                inb = (vj * 16 + lanes) < m_count
                return acc + jnp.where(inb & (cv >= txi), 1, 0)

            accv = lax.fori_loop(0, nv, cnt_body, jnp.zeros((16,), jnp.int32))
            return jnp.where(jnp.sum(accv) >= rprime, t, p)

        p_low = lax.fori_loop(0, 17, bit_body, jnp.int32(0))
        tx = (lax.shift_left(b_star, 17) | p_low) ^ sign

        @plsc.parallel_loop(0, _NCHUNK, unroll=8)
        def mask_loop(i):
            bits = plsc.bitcast(x_v[pl.ds(i * 16, 16)], jnp.int32)
            xk = jnp.where(bits < 0, bits ^ inv, bits)
            x_v[pl.ds(i * 16, 16)] = jnp.where(xk >= tx,
                                               jnp.float32(1.0),
                                               jnp.float32(0.0))
            hist_v[pl.ds(i * 16, 16)] = jnp.zeros((16,), jnp.int32)

        pltpu.sync_copy(x_v, out_hbm.at[row])
        return _

    lax.fori_loop(0, _ROWS_PER_WORKER, do_row, jnp.int32(0))


@jax.jit
def kernel(output_a):
    mesh = plsc.VectorSubcoreMesh(core_axis_name="c", subcore_axis_name="s",
                                  num_cores=2, num_subcores=16)
    fn = functools.partial(
        pl.kernel,
        out_type=jax.ShapeDtypeStruct((_NROWS, _NCOLS), jnp.float32),
        mesh=mesh,
        compiler_params=pltpu.CompilerParams(needs_layout_passes=False),
        scratch_types=[
            pltpu.VMEM((_NCOLS,), jnp.float32),
            pltpu.VMEM((_NBINS,), jnp.int32),
            pltpu.VMEM((_NCAND,), jnp.int32),
        ],
    )(_sc_body)
    return fn(output_a)
